# baseline (speedup 1.0000x reference)
import functools

import jax
import jax.numpy as jnp
import numpy as np
from jax import lax
from jax.experimental import pallas as pl
from jax.experimental.pallas import tpu as pltpu
from jax.experimental.pallas import tpu_sc as plsc

N = 10000
E = 320000
D = 128
H = 128
O = 128

NC = 2
NS = 16
NW = NC * NS
EPT = E // NW
G = 80
NCHUNK = EPT // G
NP = 10240
RPT = NP // NS
BN = 2000
L = 16


def _pre_body(x_ref, wm_ref, wu_ref, m_ref, u_ref):
    x = x_ref[...]
    m_ref[...] = jnp.dot(x, wm_ref[...], preferred_element_type=jnp.float32)
    u_ref[...] = jnp.dot(x, wu_ref[...], preferred_element_type=jnp.float32)


def _tc_pre(x, WmT, WuT):
    return pl.pallas_call(
        _pre_body,
        grid=(N // BN,),
        in_specs=[
            pl.BlockSpec((BN, D), lambda i: (i, 0)),
            pl.BlockSpec((D, H), lambda i: (0, 0)),
            pl.BlockSpec((D, H), lambda i: (0, 0)),
        ],
        out_specs=[
            pl.BlockSpec((BN, H), lambda i: (i, 0)),
            pl.BlockSpec((BN, H), lambda i: (i, 0)),
        ],
        out_shape=[jax.ShapeDtypeStruct((N, H), jnp.float32)] * 2,
    )(x, WmT, WuT)


def _mid_body(u_ref, p_ref, b_ref, wm_ref, wu_ref, m_ref, u2_ref):
    h = jnp.tanh(u_ref[...] + b_ref[...] + p_ref[0] + p_ref[1])
    m_ref[...] = jnp.dot(h, wm_ref[...], preferred_element_type=jnp.float32)
    u2_ref[...] = jnp.dot(h, wu_ref[...], preferred_element_type=jnp.float32)


def _tc_mid(u, p, b, WmT, WuT):
    return pl.pallas_call(
        _mid_body,
        grid=(N // BN,),
        in_specs=[
            pl.BlockSpec((BN, H), lambda i: (i, 0)),
            pl.BlockSpec((NC, BN, H), lambda i: (0, i, 0)),
            pl.BlockSpec((1, H), lambda i: (0, 0)),
            pl.BlockSpec((H, H), lambda i: (0, 0)),
            pl.BlockSpec((H, H), lambda i: (0, 0)),
        ],
        out_specs=[
            pl.BlockSpec((BN, H), lambda i: (i, 0)),
            pl.BlockSpec((BN, H), lambda i: (i, 0)),
        ],
        out_shape=[jax.ShapeDtypeStruct((N, H), jnp.float32)] * 2,
    )(u, p, b, WmT, WuT)


def _post_body(u_ref, p_ref, b_ref, wo_ref, bo_ref, o_ref):
    h = jnp.tanh(u_ref[...] + b_ref[...] + p_ref[0] + p_ref[1])
    o_ref[...] = (
        jnp.dot(h, wo_ref[...], preferred_element_type=jnp.float32)
        + bo_ref[...]
    )


def _tc_post(u, p, b, WoT, bo):
    return pl.pallas_call(
        _post_body,
        grid=(N // BN,),
        in_specs=[
            pl.BlockSpec((BN, H), lambda i: (i, 0)),
            pl.BlockSpec((NC, BN, H), lambda i: (0, i, 0)),
            pl.BlockSpec((1, H), lambda i: (0, 0)),
            pl.BlockSpec((H, O), lambda i: (0, 0)),
            pl.BlockSpec((1, O), lambda i: (0, 0)),
        ],
        out_specs=pl.BlockSpec((BN, O), lambda i: (i, 0)),
        out_shape=jax.ShapeDtypeStruct((N, O), jnp.float32),
    )(u, p, b, WoT, bo)


def _sc_body(m_hbm, src_hbm, dst_hbm, w_hbm, zeros_hbm, out_hbm,
             acc_sh, src_b, dst_b, w_ring, rows_b,
             esem0, esem1, ssem0, ssem1, gsem0, gsem1):
    cid = lax.axis_index("c")
    sid = lax.axis_index("s")
    tid = cid * NS + sid
    ebase = tid * EPT
    esems = [esem0, esem1]
    ssems = [ssem0, ssem1]
    gsems = [gsem0, gsem1]

    def e_start(c, sem):
        sp = lax.rem(c, 4)
        dp = lax.rem(c, 8)
        e0 = ebase + c * G
        pltpu.async_copy(src_hbm.at[pl.ds(e0, G)], src_b.at[sp], sem)
        pltpu.async_copy(dst_hbm.at[pl.ds(e0, G)], dst_b.at[dp], sem)
        pltpu.async_copy(w_hbm.at[pl.ds(e0, G)],
                         w_ring.at[pl.ds(sp * G, G)], sem)

    def e_wait(c, sem):
        sp = lax.rem(c, 4)
        dp = lax.rem(c, 8)
        e0 = ebase + c * G
        pltpu.make_async_copy(src_hbm.at[pl.ds(e0, G)], src_b.at[sp],
                              sem).wait()
        pltpu.make_async_copy(dst_hbm.at[pl.ds(e0, G)], dst_b.at[dp],
                              sem).wait()
        pltpu.make_async_copy(w_hbm.at[pl.ds(e0, G)],
                              w_ring.at[pl.ds(sp * G, G)], sem).wait()

    def g_start(c, sem):
        sp = lax.rem(c, 4)
        pltpu.async_copy(m_hbm.at[src_b.at[sp]], rows_b.at[sp], sem)

    def g_wait(c, sem):
        sp = lax.rem(c, 4)
        pltpu.make_async_copy(m_hbm.at[src_b.at[sp]], rows_b.at[sp],
                              sem).wait()

    def s_start(c, sem):
        sp = lax.rem(c, 4)
        dp = lax.rem(c, 8)
        pltpu.make_async_copy(rows_b.at[sp], acc_sh.at[dst_b.at[dp]],
                              sem).start(add=True)

    def s_wait(c, sem):
        sp = lax.rem(c, 4)
        dp = lax.rem(c, 8)
        pltpu.make_async_copy(rows_b.at[sp], acc_sh.at[dst_b.at[dp]],
                              sem).wait()

    def compute(c):
        sp = lax.rem(c, 4)
        woff = sp * G

        @plsc.parallel_loop(0, G, unroll=8)
        def _scale_edge(e):
            wv = plsc.load_gather(
                w_ring, [jnp.full((L,), woff, jnp.int32) + e])
            for k in range(H // L):
                rows_b[sp, e, pl.ds(k * L, L)] = (
                    rows_b[sp, e, pl.ds(k * L, L)] * wv)

    def step(c, p, do_swait=True, do_gnext=True, do_enext=True):
        g_wait(c, gsems[p])
        if do_swait:
            s_wait(c - 2, ssems[p])
        if do_gnext:
            e_wait(c + 2, esems[p])
            g_start(c + 2, gsems[p])
        compute(c)
        s_start(c, ssems[p])
        if do_enext:
            e_start(c + 3, esems[1 - p])

    e_start(0, esems[0])
    e_start(1, esems[1])
    pltpu.sync_copy(zeros_hbm.at[pl.ds(sid * RPT, RPT)],
                    acc_sh.at[pl.ds(sid * RPT, RPT)])
    plsc.subcore_barrier()
    e_wait(0, esems[0])
    g_start(0, gsems[0])
    e_start(2, esems[0])
    e_wait(1, esems[1])
    g_start(1, gsems[1])

    step(0, 0, do_swait=False)
    step(1, 1, do_swait=False)

    def round_body(r, carry):
        step(2 * r, 0)
        step(2 * r + 1, 1)
        return carry

    lax.fori_loop(1, (NCHUNK - 3) // 2, round_body, 0)

    step(NCHUNK - 3, 0, do_enext=False)
    step(NCHUNK - 2, 1, do_gnext=False, do_enext=False)
    step(NCHUNK - 1, 0, do_gnext=False, do_enext=False)
    s_wait(NCHUNK - 2, ssems[1])
    s_wait(NCHUNK - 1, ssems[0])

    plsc.subcore_barrier()
    pltpu.sync_copy(acc_sh.at[pl.ds(sid * RPT, RPT)],
                    out_hbm.at[cid].at[pl.ds(sid * RPT, RPT)])


def _sc_agg(m, src_g, dst_g, w_g, zeros):
    mesh = plsc.VectorSubcoreMesh(
        core_axis_name="c", subcore_axis_name="s",
        num_cores=NC, num_subcores=NS)
    f = functools.partial(
        pl.kernel,
        out_type=jax.ShapeDtypeStruct((NC, NP, H), jnp.float32),
        mesh=mesh,
        compiler_params=pltpu.CompilerParams(needs_layout_passes=False),
        scratch_types=[
            pltpu.VMEM_SHARED((NP, H), jnp.float32),
            pltpu.VMEM((4, G), jnp.int32),
            pltpu.VMEM((8, G), jnp.int32),
            pltpu.VMEM((4 * G,), jnp.float32),
            pltpu.VMEM((4, G, H), jnp.float32),
            pltpu.SemaphoreType.DMA,
            pltpu.SemaphoreType.DMA,
            pltpu.SemaphoreType.DMA,
            pltpu.SemaphoreType.DMA,
            pltpu.SemaphoreType.DMA,
            pltpu.SemaphoreType.DMA,
        ],
    )(_sc_body)
    return f(m, src_g, dst_g, w_g, zeros)


def kernel(x, edge_index, edge_weight, W_msg0, W_upd0, b_upd0,
           W_msg1, W_upd1, b_upd1, W_out, b_out):
    src_g = edge_index[0]
    dst_g = edge_index[1]
    zeros = jnp.zeros((NP, H), jnp.float32)

    b0 = b_upd0.reshape(1, H)
    b1 = b_upd1.reshape(1, H)
    bo = b_out.reshape(1, O)

    m0, u0 = _tc_pre(x, W_msg0.T, W_upd0.T)
    p0 = _sc_agg(m0, src_g, dst_g, edge_weight, zeros)
    m1, u1 = _tc_mid(u0, p0, b0, W_msg1.T, W_upd1.T)
    p1 = _sc_agg(m1, src_g, dst_g, edge_weight, zeros)
    out = _tc_post(u1, p1, b1, W_out.T, bo)
    return out


# speedup vs baseline: 1.0069x; 1.0069x over previous
import functools

import jax
import jax.numpy as jnp
import numpy as np
from jax import lax
from jax.experimental import pallas as pl
from jax.experimental.pallas import tpu as pltpu
from jax.experimental.pallas import tpu_sc as plsc

N = 10000
E = 320000
D = 128
H = 128
O = 128

NC = 2
NS = 16
NW = NC * NS
EPT = E // NW
G = 80
NCHUNK = EPT // G
NP = 10240
RPT = NP // NS
BN = 2000
L = 16


def _pre_body(x_ref, wm_ref, wu_ref, m_ref, u_ref):
    x = x_ref[...]
    m_ref[...] = jnp.dot(x, wm_ref[...], preferred_element_type=jnp.float32)
    u_ref[...] = jnp.dot(x, wu_ref[...], preferred_element_type=jnp.float32)


def _tc_pre(x, WmT, WuT):
    return pl.pallas_call(
        _pre_body,
        grid=(N // BN,),
        in_specs=[
            pl.BlockSpec((BN, D), lambda i: (i, 0)),
            pl.BlockSpec((D, H), lambda i: (0, 0)),
            pl.BlockSpec((D, H), lambda i: (0, 0)),
        ],
        out_specs=[
            pl.BlockSpec((BN, H), lambda i: (i, 0)),
            pl.BlockSpec((BN, H), lambda i: (i, 0)),
        ],
        out_shape=[jax.ShapeDtypeStruct((N, H), jnp.float32)] * 2,
    )(x, WmT, WuT)


def _mid_body(u_ref, p_ref, b_ref, wm_ref, wu_ref, m_ref, u2_ref):
    h = jnp.tanh(u_ref[...] + b_ref[...] + p_ref[0] + p_ref[1])
    m_ref[...] = jnp.dot(h, wm_ref[...], preferred_element_type=jnp.float32)
    u2_ref[...] = jnp.dot(h, wu_ref[...], preferred_element_type=jnp.float32)


def _tc_mid(u, p, b, WmT, WuT):
    return pl.pallas_call(
        _mid_body,
        grid=(N // BN,),
        in_specs=[
            pl.BlockSpec((BN, H), lambda i: (i, 0)),
            pl.BlockSpec((NC, BN, H), lambda i: (0, i, 0)),
            pl.BlockSpec((1, H), lambda i: (0, 0)),
            pl.BlockSpec((H, H), lambda i: (0, 0)),
            pl.BlockSpec((H, H), lambda i: (0, 0)),
        ],
        out_specs=[
            pl.BlockSpec((BN, H), lambda i: (i, 0)),
            pl.BlockSpec((BN, H), lambda i: (i, 0)),
        ],
        out_shape=[jax.ShapeDtypeStruct((N, H), jnp.float32)] * 2,
    )(u, p, b, WmT, WuT)


def _post_body(u_ref, p_ref, b_ref, wo_ref, bo_ref, o_ref):
    h = jnp.tanh(u_ref[...] + b_ref[...] + p_ref[0] + p_ref[1])
    o_ref[...] = (
        jnp.dot(h, wo_ref[...], preferred_element_type=jnp.float32)
        + bo_ref[...]
    )


def _tc_post(u, p, b, WoT, bo):
    return pl.pallas_call(
        _post_body,
        grid=(N // BN,),
        in_specs=[
            pl.BlockSpec((BN, H), lambda i: (i, 0)),
            pl.BlockSpec((NC, BN, H), lambda i: (0, i, 0)),
            pl.BlockSpec((1, H), lambda i: (0, 0)),
            pl.BlockSpec((H, O), lambda i: (0, 0)),
            pl.BlockSpec((1, O), lambda i: (0, 0)),
        ],
        out_specs=pl.BlockSpec((BN, O), lambda i: (i, 0)),
        out_shape=jax.ShapeDtypeStruct((N, O), jnp.float32),
    )(u, p, b, WoT, bo)


def _sc_body(m_hbm, src_hbm, dst_hbm, w_hbm, zeros_hbm, out_hbm,
             acc_sh, src_b, dst_b, w_ring, rows_b,
             esem0, esem1, ssem0, ssem1, gsem0, gsem1):
    cid = lax.axis_index("c")
    sid = lax.axis_index("s")
    tid = cid * NS + sid
    ebase = tid * EPT
    esems = [esem0, esem1]
    ssems = [ssem0, ssem1]
    gsems = [gsem0, gsem1]

    def e_start(c, sem):
        sp = lax.rem(c, 4)
        dp = lax.rem(c, 8)
        e0 = ebase + c * G
        pltpu.async_copy(src_hbm.at[pl.ds(e0, G)], src_b.at[sp], sem)
        pltpu.async_copy(dst_hbm.at[pl.ds(e0, G)], dst_b.at[dp], sem)
        pltpu.async_copy(w_hbm.at[pl.ds(e0, G)],
                         w_ring.at[pl.ds(sp * G, G)], sem)

    def e_wait(c, sem):
        sp = lax.rem(c, 4)
        dp = lax.rem(c, 8)
        e0 = ebase + c * G
        pltpu.make_async_copy(src_hbm.at[pl.ds(e0, G)], src_b.at[sp],
                              sem).wait()
        pltpu.make_async_copy(dst_hbm.at[pl.ds(e0, G)], dst_b.at[dp],
                              sem).wait()
        pltpu.make_async_copy(w_hbm.at[pl.ds(e0, G)],
                              w_ring.at[pl.ds(sp * G, G)], sem).wait()

    def g_start(c, sem):
        sp = lax.rem(c, 4)
        pltpu.async_copy(m_hbm.at[src_b.at[sp]], rows_b.at[sp], sem)

    def g_wait(c, sem):
        sp = lax.rem(c, 4)
        pltpu.make_async_copy(m_hbm.at[src_b.at[sp]], rows_b.at[sp],
                              sem).wait()

    def s_start(c, sem):
        sp = lax.rem(c, 4)
        dp = lax.rem(c, 8)
        pltpu.make_async_copy(rows_b.at[sp], acc_sh.at[dst_b.at[dp]],
                              sem).start(add=True)

    def s_wait(c, sem):
        sp = lax.rem(c, 4)
        dp = lax.rem(c, 8)
        pltpu.make_async_copy(rows_b.at[sp], acc_sh.at[dst_b.at[dp]],
                              sem).wait()

    def compute(c):
        sp = lax.rem(c, 4)
        woff = sp * G

        @plsc.parallel_loop(0, G, unroll=4,
                            carry=jnp.full((L,), woff, jnp.int32))
        def _scale_edge(e, ev):
            wv = plsc.load_gather(w_ring, [ev])
            for k in range(H // L):
                rows_b[sp, e, pl.ds(k * L, L)] = (
                    rows_b[sp, e, pl.ds(k * L, L)] * wv)
            return ev + 1

    def step(c, p, do_swait=True, do_gnext=True, do_enext=True):
        g_wait(c, gsems[p])
        if do_swait:
            s_wait(c - 2, ssems[p])
        if do_gnext:
            e_wait(c + 2, esems[p])
            g_start(c + 2, gsems[p])
        compute(c)
        s_start(c, ssems[p])
        if do_enext:
            e_start(c + 3, esems[1 - p])

    e_start(0, esems[0])
    e_start(1, esems[1])
    pltpu.sync_copy(zeros_hbm.at[pl.ds(sid * RPT, RPT)],
                    acc_sh.at[pl.ds(sid * RPT, RPT)])
    plsc.subcore_barrier()
    e_wait(0, esems[0])
    g_start(0, gsems[0])
    e_start(2, esems[0])
    e_wait(1, esems[1])
    g_start(1, gsems[1])

    step(0, 0, do_swait=False)
    step(1, 1, do_swait=False)

    def round_body(r, carry):
        step(2 * r, 0)
        step(2 * r + 1, 1)
        return carry

    lax.fori_loop(1, (NCHUNK - 3) // 2, round_body, 0)

    step(NCHUNK - 3, 0, do_enext=False)
    step(NCHUNK - 2, 1, do_gnext=False, do_enext=False)
    step(NCHUNK - 1, 0, do_gnext=False, do_enext=False)
    s_wait(NCHUNK - 2, ssems[1])
    s_wait(NCHUNK - 1, ssems[0])

    plsc.subcore_barrier()
    pltpu.sync_copy(acc_sh.at[pl.ds(sid * RPT, RPT)],
                    out_hbm.at[cid].at[pl.ds(sid * RPT, RPT)])


def _sc_agg(m, src_g, dst_g, w_g, zeros):
    mesh = plsc.VectorSubcoreMesh(
        core_axis_name="c", subcore_axis_name="s",
        num_cores=NC, num_subcores=NS)
    f = functools.partial(
        pl.kernel,
        out_type=jax.ShapeDtypeStruct((NC, NP, H), jnp.float32),
        mesh=mesh,
        compiler_params=pltpu.CompilerParams(needs_layout_passes=False),
        scratch_types=[
            pltpu.VMEM_SHARED((NP, H), jnp.float32),
            pltpu.VMEM((4, G), jnp.int32),
            pltpu.VMEM((8, G), jnp.int32),
            pltpu.VMEM((4 * G,), jnp.float32),
            pltpu.VMEM((4, G, H), jnp.float32),
            pltpu.SemaphoreType.DMA,
            pltpu.SemaphoreType.DMA,
            pltpu.SemaphoreType.DMA,
            pltpu.SemaphoreType.DMA,
            pltpu.SemaphoreType.DMA,
            pltpu.SemaphoreType.DMA,
        ],
    )(_sc_body)
    return f(m, src_g, dst_g, w_g, zeros)


def kernel(x, edge_index, edge_weight, W_msg0, W_upd0, b_upd0,
           W_msg1, W_upd1, b_upd1, W_out, b_out):
    src_g = edge_index[0]
    dst_g = edge_index[1]
    zeros = jnp.zeros((NP, H), jnp.float32)

    b0 = b_upd0.reshape(1, H)
    b1 = b_upd1.reshape(1, H)
    bo = b_out.reshape(1, O)

    m0, u0 = _tc_pre(x, W_msg0.T, W_upd0.T)
    p0 = _sc_agg(m0, src_g, dst_g, edge_weight, zeros)
    m1, u1 = _tc_mid(u0, p0, b0, W_msg1.T, W_upd1.T)
    p1 = _sc_agg(m1, src_g, dst_g, edge_weight, zeros)
    out = _tc_post(u1, p1, b1, W_out.T, bo)
    return out
